# baseline (speedup 1.0000x reference)
import functools

import jax
import jax.numpy as jnp
from jax.experimental import pallas as pl
from jax.experimental.pallas import tpu as pltpu


def _eca_nhwc_kernel(nb, w_ref, x_ref, o_ref):
    for i in range(nb):
        x = x_ref[i]
        s = jnp.sum(x, axis=(0, 1), keepdims=True)
        z = jnp.zeros((1, 1, 1), dtype=s.dtype)
        s_lo = jnp.concatenate([z, s[:, :, :-1]], axis=2)
        s_hi = jnp.concatenate([s[:, :, 1:], z], axis=2)
        att = jax.nn.sigmoid(w_ref[0] * s_lo + w_ref[1] * s + w_ref[2] * s_hi)
        o_ref[i] = x * att.astype(x.dtype)


def kernel(x_nchw, conv_weight):
    B, C, H, W = x_nchw.shape
    HW = H * W
    dtype = x_nchw.dtype
    itemsize = jnp.dtype(dtype).itemsize

    w = conv_weight.reshape(-1).astype(jnp.float32)
    assert w.shape[0] == 3, "specialized for k_size=3"
    w = w * (1.0 / float(HW))

    x_bhwc = jnp.transpose(x_nchw, (0, 2, 3, 1))

    nb = 4 if B % 4 == 0 else (2 if B % 2 == 0 else 1)

    out_bhwc = pl.pallas_call(
        functools.partial(_eca_nhwc_kernel, nb),
        out_shape=jax.ShapeDtypeStruct((B, H, W, C), dtype),
        grid=(B // nb,),
        in_specs=[
            pl.BlockSpec(memory_space=pltpu.SMEM),
            pl.BlockSpec((nb, H, W, C), lambda b: (b, 0, 0, 0)),
        ],
        out_specs=pl.BlockSpec((nb, H, W, C), lambda b: (b, 0, 0, 0)),
        compiler_params=pltpu.CompilerParams(
            dimension_semantics=("arbitrary",),
            vmem_limit_bytes=56 * 1024 * 1024,
        ),
        cost_estimate=pl.CostEstimate(
            flops=int(2 * B * C * HW + 8 * B * C),
            transcendentals=int(B * C),
            bytes_accessed=int(2 * B * C * HW * itemsize),
        ),
    )(w, x_bhwc)
    return jnp.transpose(out_bhwc, (0, 3, 1, 2))


# speedup vs baseline: 1.0014x; 1.0014x over previous
import functools

import jax
import jax.numpy as jnp
from jax.experimental import pallas as pl
from jax.experimental.pallas import tpu as pltpu


def _eca_nhwc_kernel(nb, w_ref, x_ref, o_ref):
    for i in range(nb):
        x = x_ref[i]
        s = jnp.sum(x, axis=(0, 1), keepdims=True)
        z = jnp.zeros((1, 1, 1), dtype=s.dtype)
        s_lo = jnp.concatenate([z, s[:, :, :-1]], axis=2)
        s_hi = jnp.concatenate([s[:, :, 1:], z], axis=2)
        att = jax.nn.sigmoid(w_ref[0] * s_lo + w_ref[1] * s + w_ref[2] * s_hi)
        o_ref[i] = x * att.astype(x.dtype)


def kernel(x_nchw, conv_weight):
    B, C, H, W = x_nchw.shape
    HW = H * W
    dtype = x_nchw.dtype
    itemsize = jnp.dtype(dtype).itemsize

    w = conv_weight.reshape(-1).astype(jnp.float32)
    assert w.shape[0] == 3, "specialized for k_size=3"
    w = w * (1.0 / float(HW))

    x_bhwc = jnp.transpose(x_nchw, (0, 2, 3, 1))

    nb = 4 if B % 4 == 0 else (2 if B % 2 == 0 else 1)

    out_bhwc = pl.pallas_call(
        functools.partial(_eca_nhwc_kernel, nb),
        out_shape=jax.ShapeDtypeStruct((B, H, W, C), dtype),
        grid=(B // nb,),
        in_specs=[
            pl.BlockSpec(memory_space=pltpu.SMEM),
            pl.BlockSpec((nb, H, W, C), lambda b: (b, 0, 0, 0)),
        ],
        out_specs=pl.BlockSpec((nb, H, W, C), lambda b: (b, 0, 0, 0)),
        compiler_params=pltpu.CompilerParams(
            dimension_semantics=("parallel",),
            vmem_limit_bytes=56 * 1024 * 1024,
        ),
        cost_estimate=pl.CostEstimate(
            flops=int(2 * B * C * HW + 8 * B * C),
            transcendentals=int(B * C),
            bytes_accessed=int(2 * B * C * HW * itemsize),
        ),
    )(w, x_bhwc)
    return jnp.transpose(out_bhwc, (0, 3, 1, 2))


# speedup vs baseline: 1.0016x; 1.0001x over previous
import functools

import jax
import jax.numpy as jnp
from jax.experimental import pallas as pl
from jax.experimental.pallas import tpu as pltpu


def _eca_nhwc_kernel(nb, w_ref, x_ref, o_ref):
    for i in range(nb):
        x = x_ref[i]
        s = jnp.sum(x, axis=(0, 1), keepdims=True)
        z = jnp.zeros((1, 1, 1), dtype=s.dtype)
        s_lo = jnp.concatenate([z, s[:, :, :-1]], axis=2)
        s_hi = jnp.concatenate([s[:, :, 1:], z], axis=2)
        att = jax.nn.sigmoid(w_ref[0] * s_lo + w_ref[1] * s + w_ref[2] * s_hi)
        o_ref[i] = x * att.astype(x.dtype)


def kernel(x_nchw, conv_weight):
    B, C, H, W = x_nchw.shape
    HW = H * W
    dtype = x_nchw.dtype
    itemsize = jnp.dtype(dtype).itemsize

    w = conv_weight.reshape(-1).astype(jnp.float32)
    assert w.shape[0] == 3, "specialized for k_size=3"
    w = w * (1.0 / float(HW))

    x_bhwc = jnp.transpose(x_nchw, (0, 2, 3, 1))

    img_bytes = H * W * C * itemsize
    nb = 1
    for cand in (4, 2):
        if B % cand == 0 and 4 * cand * img_bytes <= 52 * 1024 * 1024:
            nb = cand
            break

    out_bhwc = pl.pallas_call(
        functools.partial(_eca_nhwc_kernel, nb),
        out_shape=jax.ShapeDtypeStruct((B, H, W, C), dtype),
        grid=(B // nb,),
        in_specs=[
            pl.BlockSpec(memory_space=pltpu.SMEM),
            pl.BlockSpec((nb, H, W, C), lambda b: (b, 0, 0, 0)),
        ],
        out_specs=pl.BlockSpec((nb, H, W, C), lambda b: (b, 0, 0, 0)),
        compiler_params=pltpu.CompilerParams(
            dimension_semantics=("parallel",),
            vmem_limit_bytes=56 * 1024 * 1024,
        ),
        cost_estimate=pl.CostEstimate(
            flops=int(2 * B * C * HW + 8 * B * C),
            transcendentals=int(B * C),
            bytes_accessed=int(2 * B * C * HW * itemsize),
        ),
    )(w, x_bhwc)
    return jnp.transpose(out_bhwc, (0, 3, 1, 2))
